# baseline (speedup 1.0000x reference)
import functools

import jax
import jax.numpy as jnp
from jax.experimental import pallas as pl
from jax.experimental.pallas import tpu as pltpu

_NCHUNKS = 8


def _dma_copy_body(x_ref, o_ref, *sems):
    rows = x_ref.shape[0]
    chunk = rows // _NCHUNKS
    copies = []
    for i in range(_NCHUNKS):
        sl = pl.ds(i * chunk, chunk)
        c = pltpu.make_async_copy(x_ref.at[sl], o_ref.at[sl], sems[i])
        c.start()
        copies.append(c)
    for c in copies:
        c.wait()


def kernel(hidden_state):
    B, S, D = hidden_state.shape
    x = hidden_state.reshape(B * S, D)
    rows = B * S
    out = pl.pallas_call(
        _dma_copy_body,
        in_specs=[pl.BlockSpec(memory_space=pl.ANY)],
        out_specs=pl.BlockSpec(memory_space=pl.ANY),
        out_shape=jax.ShapeDtypeStruct((rows, D), hidden_state.dtype),
        scratch_shapes=[pltpu.SemaphoreType.DMA] * _NCHUNKS,
    )(x)
    return out.reshape(B, S, D)


# speedup vs baseline: 48.4759x; 48.4759x over previous
import jax
import jax.numpy as jnp
from jax.experimental import pallas as pl
from jax.experimental.pallas import tpu as pltpu


def _copy_body(x_ref, o_ref):
    o_ref[...] = x_ref[...]


def kernel(hidden_state):
    B, S, D = hidden_state.shape
    x = hidden_state.reshape(B * S, D)
    rows = B * S
    block_rows = 512
    grid = (rows // block_rows,)
    out = pl.pallas_call(
        _copy_body,
        grid=grid,
        in_specs=[pl.BlockSpec((block_rows, D), lambda i: (i, 0))],
        out_specs=pl.BlockSpec((block_rows, D), lambda i: (i, 0)),
        out_shape=jax.ShapeDtypeStruct((rows, D), hidden_state.dtype),
        compiler_params=pltpu.CompilerParams(
            vmem_limit_bytes=100 * 1024 * 1024,
        ),
    )(x)
    return out.reshape(B, S, D)


# speedup vs baseline: 49.3372x; 1.0178x over previous
import jax
import jax.numpy as jnp
from jax.experimental import pallas as pl
from jax.experimental.pallas import tpu as pltpu


def _copy_body(x_ref, o_ref):
    o_ref[...] = x_ref[...]


def kernel(hidden_state):
    B, S, D = hidden_state.shape
    x = hidden_state.reshape(B * S, D)
    rows = B * S
    block_rows = 2040
    grid = (pl.cdiv(rows, block_rows),)
    out = pl.pallas_call(
        _copy_body,
        grid=grid,
        in_specs=[pl.BlockSpec((block_rows, D), lambda i: (i, 0))],
        out_specs=pl.BlockSpec((block_rows, D), lambda i: (i, 0)),
        out_shape=jax.ShapeDtypeStruct((rows, D), hidden_state.dtype),
        compiler_params=pltpu.CompilerParams(
            vmem_limit_bytes=100 * 1024 * 1024,
        ),
    )(x)
    return out.reshape(B, S, D)
